# baseline (speedup 1.0000x reference)
import functools

import jax
import jax.numpy as jnp
from jax import lax
from jax.experimental import pallas as pl
from jax.experimental.pallas import tpu as pltpu
from jax.experimental.pallas import tpu_sc as plsc

H = 512
W = 512
NPTS = 2048
MIN_DEPTH = 0.1
MAX_DEPTH = 2.0
FX = 525.0
FY = 525.0
CX = (W - 1) / 2.0
CY = (H - 1) / 2.0
BIG = 1 << 30

_SC_INFO = plsc.get_sparse_core_info()
_NC = _SC_INFO.num_cores
_NS = _SC_INFO.num_subcores
_NW = _NC * _NS
_PPW = NPTS // _NW


def _fps_body(depth_ref, idx_ref, sxyz_ref, px, py, pz, dist):
    depth = depth_ref[...]
    u = lax.broadcasted_iota(jnp.int32, (H, W), 1).astype(jnp.float32)
    v = lax.broadcasted_iota(jnp.int32, (H, W), 0).astype(jnp.float32)
    x = (u - CX) * depth / FX
    y = (v - CY) * depth / FY
    finite = (depth - depth) == 0.0
    valid = (depth > MIN_DEPTH) & (depth < MAX_DEPTH) & (depth > 0.0) & finite
    px[...] = jnp.where(valid, x, 0.0)
    py[...] = jnp.where(valid, y, 0.0)
    pz[...] = jnp.where(valid, depth, 0.0)
    dist[...] = jnp.where(valid, 1e38, -1e38)

    colio = lax.broadcasted_iota(jnp.int32, (H, W), 1)
    rowio = lax.broadcasted_iota(jnp.int32, (H, 1), 0)
    colio1 = lax.broadcasted_iota(jnp.int32, (1, W), 1)

    colcand = jnp.where(valid, colio, BIG)
    rowmin = jnp.min(colcand, axis=1, keepdims=True)
    rcand = jnp.where(rowmin < BIG, rowio, BIG)
    r0 = jnp.min(rcand)
    r0 = jnp.where(r0 < BIG, r0, 0)
    dr0 = depth_ref[pl.ds(r0, 1), :]
    fin0 = (dr0 - dr0) == 0.0
    val0 = (dr0 > MIN_DEPTH) & (dr0 < MAX_DEPTH) & (dr0 > 0.0) & fin0
    c0 = jnp.min(jnp.where(val0, colio1, BIG))
    c0 = jnp.where(c0 < BIG, c0, 0)

    def body(i, rc):
        r, c = rc
        pxr = px[pl.ds(r, 1), :]
        pyr = py[pl.ds(r, 1), :]
        pzr = pz[pl.ds(r, 1), :]
        sel = colio1 == c
        cxs = jnp.sum(jnp.where(sel, pxr, 0.0))
        cys = jnp.sum(jnp.where(sel, pyr, 0.0))
        czs = jnp.sum(jnp.where(sel, pzr, 0.0))
        idx_ref[i] = r * W + c
        sxyz_ref[0, i] = cxs
        sxyz_ref[1, i] = cys
        sxyz_ref[2, i] = czs
        dx = px[...] - cxs
        dy = py[...] - cys
        dz = pz[...] - czs
        d = dx * dx + dy * dy + dz * dz
        nd = jnp.minimum(dist[...], d)
        dist[...] = nd
        rowmax = jnp.max(nd, axis=1, keepdims=True)
        m = jnp.max(rowmax)
        r2 = jnp.min(jnp.where(rowmax == m, rowio, BIG))
        drow = dist[pl.ds(r2, 1), :]
        c2 = jnp.min(jnp.where(drow == m, colio1, BIG))
        return (r2, c2)

    lax.fori_loop(0, NPTS, body, (r0, c0))


def _rgb_gather_body(rgb_hbm, idx_hbm, out_hbm, idx_v, idx3_v, rows_v, sem):
    wid = lax.axis_index("s") * _NC + lax.axis_index("c")
    base = wid * _PPW
    pltpu.sync_copy(idx_hbm.at[pl.ds(base, _PPW)], idx_v)
    lane = lax.iota(jnp.int32, 16)
    for ch in range(3):
        for k in range(_PPW // 16):
            p = idx_v[pl.ds(16 * k, 16)]
            idx3_v[pl.ds(16 * k, 16)] = p * 3 + ch
        pltpu.async_copy(rgb_hbm.at[idx3_v], rows_v, sem).wait()
        pltpu.sync_copy(rows_v, out_hbm.at[ch, pl.ds(base, _PPW)])
    del lane


_rgb_gather = functools.partial(
    pl.kernel,
    mesh=plsc.VectorSubcoreMesh(core_axis_name="c", subcore_axis_name="s"),
    out_type=jax.ShapeDtypeStruct((3, NPTS), jnp.float32),
    scratch_types=[
        pltpu.VMEM((_PPW,), jnp.int32),
        pltpu.VMEM((_PPW,), jnp.int32),
        pltpu.VMEM((_PPW,), jnp.float32),
        pltpu.SemaphoreType.DMA,
    ],
)(_rgb_gather_body)


def _assemble_body(sxyz_ref, srgb_ref, out_ref):
    s = sxyz_ref[...]
    rgb = srgb_ref[...] / 255.0
    mn = jnp.min(s, axis=1, keepdims=True)
    centered = s - mn
    mx = jnp.max(centered, axis=1, keepdims=True)
    mx = jnp.where(mx < 1e-8, 1.0, mx)
    out_ref[...] = jnp.concatenate([s, rgb, centered / mx], axis=0)


def kernel(depth_image, rgb_image, key):
    idx, sxyz = pl.pallas_call(
        _fps_body,
        out_shape=[
            jax.ShapeDtypeStruct((NPTS,), jnp.int32),
            jax.ShapeDtypeStruct((3, NPTS), jnp.float32),
        ],
        in_specs=[pl.BlockSpec(memory_space=pltpu.VMEM)],
        out_specs=[
            pl.BlockSpec(memory_space=pltpu.SMEM),
            pl.BlockSpec(memory_space=pltpu.SMEM),
        ],
        scratch_shapes=[pltpu.VMEM((H, W), jnp.float32)] * 4,
    )(depth_image)
    srgb = _rgb_gather(rgb_image.reshape(-1), idx)
    out_t = pl.pallas_call(
        _assemble_body,
        out_shape=jax.ShapeDtypeStruct((9, NPTS), jnp.float32),
    )(sxyz, srgb)
    return out_t.T


# speedup vs baseline: 1.1242x; 1.1242x over previous
import functools

import jax
import jax.numpy as jnp
from jax import lax
from jax.experimental import pallas as pl
from jax.experimental.pallas import tpu as pltpu
from jax.experimental.pallas import tpu_sc as plsc

H = 512
W = 512
NPTS = 2048
MIN_DEPTH = 0.1
MAX_DEPTH = 2.0
FX = 525.0
FY = 525.0
CX = (W - 1) / 2.0
CY = (H - 1) / 2.0
BIG = 1 << 30

_NC = 2
_NS = 16
_NW = _NC * _NS
_PPW = NPTS // _NW


def _fps_body(depth_ref, idx_ref, sx_ref, sy_ref, sz_ref, px, py, pz, dist):
    depth = depth_ref[...]
    u = lax.broadcasted_iota(jnp.int32, (H, W), 1).astype(jnp.float32)
    v = lax.broadcasted_iota(jnp.int32, (H, W), 0).astype(jnp.float32)
    x = (u - CX) * depth / FX
    y = (v - CY) * depth / FY
    finite = (depth - depth) == 0.0
    valid = (depth > MIN_DEPTH) & (depth < MAX_DEPTH) & (depth > 0.0) & finite
    px[...] = jnp.where(valid, x, 0.0)
    py[...] = jnp.where(valid, y, 0.0)
    pz[...] = jnp.where(valid, depth, 0.0)
    dist[...] = jnp.where(valid, 1e38, -1e38)

    colio = lax.broadcasted_iota(jnp.int32, (H, W), 1)
    rowio = lax.broadcasted_iota(jnp.int32, (H, 1), 0)
    colio1 = lax.broadcasted_iota(jnp.int32, (1, W), 1)

    colcand = jnp.where(valid, colio, BIG)
    rowmin = jnp.min(colcand, axis=1, keepdims=True)
    r0v = jnp.min(jnp.where(rowmin < BIG, rowio, BIG), axis=0, keepdims=True)
    r0v = jnp.where(r0v < BIG, r0v, 0)
    c0v = jnp.min(jnp.where(rowio == r0v, rowmin, BIG), axis=0, keepdims=True)
    c0v = jnp.where(c0v < BIG, c0v, 0)
    r0 = r0v[0, 0]

    def body(i, carry):
        r, rv, cv = carry
        pxr = px[pl.ds(r, 1), :]
        pyr = py[pl.ds(r, 1), :]
        pzr = pz[pl.ds(r, 1), :]
        sel = colio1 == cv
        cxv = jnp.sum(jnp.where(sel, pxr, 0.0), axis=1, keepdims=True)
        cyv = jnp.sum(jnp.where(sel, pyr, 0.0), axis=1, keepdims=True)
        czv = jnp.sum(jnp.where(sel, pzr, 0.0), axis=1, keepdims=True)
        idx_ref[pl.ds(i, 1), :] = rv * W + cv
        sx_ref[pl.ds(i, 1), :] = cxv
        sy_ref[pl.ds(i, 1), :] = cyv
        sz_ref[pl.ds(i, 1), :] = czv
        dx = px[...] - cxv
        dy = py[...] - cyv
        dz = pz[...] - czv
        d = dx * dx + dy * dy + dz * dz
        nd = jnp.minimum(dist[...], d)
        dist[...] = nd
        rowmax = jnp.max(nd, axis=1, keepdims=True)
        m = jnp.max(rowmax, axis=0, keepdims=True)
        r2v = jnp.min(jnp.where(rowmax == m, rowio, BIG), axis=0, keepdims=True)
        r2 = r2v[0, 0]
        drow = dist[pl.ds(r2, 1), :]
        c2v = jnp.min(jnp.where(drow == m, colio1, BIG), axis=1, keepdims=True)
        return (r2, r2v, c2v)

    lax.fori_loop(0, NPTS, body, (r0, r0v, c0v))


def _rgb_gather_body(rgb_hbm, idx_hbm, out_hbm, idx_v, idx3_v, rows_v, sem):
    wid = lax.axis_index("s") * _NC + lax.axis_index("c")
    base = wid * _PPW
    pltpu.sync_copy(idx_hbm.at[pl.ds(base, _PPW)], idx_v)
    for ch in range(3):
        for k in range(_PPW // 16):
            p = idx_v[pl.ds(16 * k, 16)]
            idx3_v[pl.ds(16 * k, 16)] = p * 3 + ch
        pltpu.async_copy(rgb_hbm.at[idx3_v], rows_v, sem).wait()
        pltpu.sync_copy(rows_v, out_hbm.at[ch, pl.ds(base, _PPW)])


@functools.cache
def _rgb_gather():
    return pl.kernel(
        _rgb_gather_body,
        mesh=plsc.VectorSubcoreMesh(core_axis_name="c", subcore_axis_name="s"),
        out_type=jax.ShapeDtypeStruct((3, NPTS), jnp.float32),
        scratch_types=[
            pltpu.VMEM((_PPW,), jnp.int32),
            pltpu.VMEM((_PPW,), jnp.int32),
            pltpu.VMEM((_PPW,), jnp.float32),
            pltpu.SemaphoreType.DMA,
        ],
    )


def _assemble_body(sx_ref, sy_ref, sz_ref, srgb_ref, out_ref):
    s = jnp.concatenate([sx_ref[...], sy_ref[...], sz_ref[...]], axis=1)
    rgb = srgb_ref[...] / 255.0
    mn = jnp.min(s, axis=0, keepdims=True)
    centered = s - mn
    mx = jnp.max(centered, axis=0, keepdims=True)
    mx = jnp.where(mx < 1e-8, 1.0, mx)
    out_ref[...] = jnp.concatenate([s, rgb, centered / mx], axis=1)


def kernel(depth_image, rgb_image, key):
    idx, sx, sy, sz = pl.pallas_call(
        _fps_body,
        out_shape=[
            jax.ShapeDtypeStruct((NPTS, 1), jnp.int32),
            jax.ShapeDtypeStruct((NPTS, 1), jnp.float32),
            jax.ShapeDtypeStruct((NPTS, 1), jnp.float32),
            jax.ShapeDtypeStruct((NPTS, 1), jnp.float32),
        ],
        in_specs=[pl.BlockSpec(memory_space=pltpu.VMEM)],
        out_specs=[pl.BlockSpec(memory_space=pltpu.VMEM)] * 4,
        scratch_shapes=[pltpu.VMEM((H, W), jnp.float32)] * 4,
    )(depth_image)
    srgb = _rgb_gather()(rgb_image.reshape(-1), idx.reshape(-1))
    out = pl.pallas_call(
        _assemble_body,
        out_shape=jax.ShapeDtypeStruct((NPTS, 9), jnp.float32),
    )(sx, sy, sz, srgb.T)
    return out
